# Initial kernel scaffold; baseline (speedup 1.0000x reference)
import jax
import jax.numpy as jnp
from jax.experimental import pallas as pl


def kernel(x, edge_index_r0, edge_index_r1, edge_index_r2, W_self_r0, W_neigh_r0, b_r0, W_self_r1, W_neigh_r1, b_r1, W_self_r2, W_neigh_r2, b_r2):
    raise NotImplementedError("write your pallas kernel here")


# speedup vs baseline: 6.0047x; 6.0047x over previous
import functools

import jax
import jax.numpy as jnp
from jax import lax
from jax.experimental import pallas as pl
from jax.experimental.pallas import tpu as pltpu
from jax.experimental.pallas import tpu_sc as plsc

_N = 10000
_D = 128
_E = 320000
_R = 3
_K = 128
_NCH = _E // _K
_NC = 2
_NS = 16
_NW = _NC * _NS
_CPT = -(-_NCH // _NW)
_ZR = 400
_CQ = _N // _ZR


def _sc_aggregate(x, srcs, dsts, zeros):
    mesh = plsc.VectorSubcoreMesh(core_axis_name="c", subcore_axis_name="s")

    @functools.partial(
        pl.kernel,
        mesh=mesh,
        out_type=(
            jax.ShapeDtypeStruct((_R, _NC, _N, _D), jnp.float32),
            jax.ShapeDtypeStruct((_R * _CQ * _NW * _ZR,), jnp.float32),
        ),
        scratch_types=[
            pltpu.VMEM((_K,), jnp.int32),
            pltpu.VMEM((_K,), jnp.int32),
            pltpu.VMEM((_K, _D), jnp.float32),
            pltpu.VMEM((_N,), jnp.float32),
            pltpu.VMEM_SHARED((_N, _D), jnp.float32),
            pltpu.SemaphoreType.DMA,
        ],
        compiler_params=pltpu.CompilerParams(needs_layout_passes=False),
    )
    def body(x_hbm, srcs_hbm, dsts_hbm, zeros_hbm, agg_out, deg_out,
             src_v, dst_v, rows_v, deg_v, agg_sh, sem):
        c = lax.axis_index("c")
        s = lax.axis_index("s")
        wid = s * _NC + c
        z16 = jnp.zeros((16,), jnp.float32)
        one16 = jnp.ones((16,), jnp.float32)

        for r in range(_R):
            for cq in range(2):
                q = cq * _NS + s

                @pl.when(q < _CQ)
                def _():
                    row = pl.multiple_of(q * _ZR, _ZR)
                    pltpu.sync_copy(zeros_hbm, agg_sh.at[pl.ds(row, _ZR)])

            def _zero_deg(i, carry):
                deg_v[pl.ds(i * 16, 16)] = z16
                return carry

            lax.fori_loop(0, _N // 16, _zero_deg, 0)
            plsc.subcore_barrier()

            def _chunk(k, carry):
                ch = k * _NW + wid

                @pl.when(ch < _NCH)
                def _():
                    base = pl.multiple_of(r * _E + ch * _K, _K)
                    pltpu.sync_copy(srcs_hbm.at[pl.ds(base, _K)], src_v)
                    pltpu.sync_copy(dsts_hbm.at[pl.ds(base, _K)], dst_v)
                    pltpu.async_copy(x_hbm.at[src_v], rows_v, sem).wait()
                    for j in range(_K // 16):
                        dj = dst_v[pl.ds(j * 16, 16)]
                        plsc.addupdate_scatter(deg_v, [dj], one16)
                    pltpu.sync_copy(rows_v, agg_sh.at[dst_v], add=True)

                return carry

            lax.fori_loop(0, _CPT, _chunk, 0)
            plsc.subcore_barrier()
            for cq in range(2):
                q = cq * _NS + s

                @pl.when(q < _CQ)
                def _():
                    row = pl.multiple_of(q * _ZR, _ZR)
                    pltpu.sync_copy(agg_sh.at[pl.ds(row, _ZR)],
                                    agg_out.at[r, c, pl.ds(row, _ZR)])

            for q in range(_CQ):
                off = pl.multiple_of(((r * _CQ + q) * _NW + wid) * _ZR, 8)
                pltpu.sync_copy(deg_v.at[pl.ds(q * _ZR, _ZR)],
                                deg_out.at[pl.ds(off, _ZR)])

    return body(x, srcs, dsts, zeros)


_BN = 400
_NB = _N // _BN


def _tc_dense(x, agg, deg, w_self, w_neigh, bias):
    def body(x_ref, agg_ref, deg_ref, ws_ref, wn_ref, b_ref, out_ref):
        xb = x_ref[...]
        ws = ws_ref[...]
        wn = wn_ref[...]
        b = b_ref[...]
        acc = jnp.dot(xb, ws[0] + ws[1] + ws[2],
                      preferred_element_type=jnp.float32)
        degs = jnp.sum(deg_ref[...], axis=(1, 2))
        inv = 1.0 / jnp.maximum(degs, 1.0)
        ag = agg_ref[...]
        for r in range(_R):
            h = (ag[r, 0] + ag[r, 1]) * inv[r][:, None]
            acc = acc + jnp.dot(h, wn[r], preferred_element_type=jnp.float32)
        acc = acc + (b[0] + b[1] + b[2])[None, :]
        out_ref[...] = acc

    return pl.pallas_call(
        body,
        grid=(_NB,),
        in_specs=[
            pl.BlockSpec((_BN, _D), lambda i: (i, 0)),
            pl.BlockSpec((_R, _NC, _BN, _D), lambda i: (0, 0, i, 0)),
            pl.BlockSpec((_R, 1, _NW, _ZR), lambda i: (0, i, 0, 0)),
            pl.BlockSpec((_R, _D, _D), lambda i: (0, 0, 0)),
            pl.BlockSpec((_R, _D, _D), lambda i: (0, 0, 0)),
            pl.BlockSpec((_R, _D), lambda i: (0, 0)),
        ],
        out_specs=pl.BlockSpec((_BN, _D), lambda i: (i, 0)),
        out_shape=jax.ShapeDtypeStruct((_N, _D), jnp.float32),
    )(x, agg, deg, w_self, w_neigh, bias)


def kernel(x, edge_index_r0, edge_index_r1, edge_index_r2,
           W_self_r0, W_neigh_r0, b_r0,
           W_self_r1, W_neigh_r1, b_r1,
           W_self_r2, W_neigh_r2, b_r2):
    srcs = jnp.concatenate(
        [edge_index_r0[0], edge_index_r1[0], edge_index_r2[0]])
    dsts = jnp.concatenate(
        [edge_index_r0[1], edge_index_r1[1], edge_index_r2[1]])
    zeros = jnp.zeros((_ZR, _D), jnp.float32)
    agg, deg_flat = _sc_aggregate(x, srcs, dsts, zeros)
    deg = deg_flat.reshape(_R, _CQ, _NW, _ZR)
    w_self = jnp.stack([W_self_r0, W_self_r1, W_self_r2])
    w_neigh = jnp.stack([W_neigh_r0, W_neigh_r1, W_neigh_r2])
    bias = jnp.stack([b_r0, b_r1, b_r2])
    return _tc_dense(x, agg, deg, w_self, w_neigh, bias)


# speedup vs baseline: 10.1884x; 1.6968x over previous
import functools

import jax
import jax.numpy as jnp
from jax import lax
from jax.experimental import pallas as pl
from jax.experimental.pallas import tpu as pltpu
from jax.experimental.pallas import tpu_sc as plsc

_N = 10000
_D = 128
_E = 320000
_R = 3
_K = 128
_NC = 2
_NS = 16
_NW = _NC * _NS
_EPT = _E // _NW
_FC = _EPT // _K
_REM = _EPT - _FC * _K
_NG = _FC // 2
_ZR = 400
_CQ = _N // _ZR


def _sc_aggregate(x, srcs, dsts, zeros):
    mesh = plsc.VectorSubcoreMesh(core_axis_name="c", subcore_axis_name="s")

    @functools.partial(
        pl.kernel,
        mesh=mesh,
        out_type=(
            jax.ShapeDtypeStruct((_R, _NC, _N, _D), jnp.float32),
            jax.ShapeDtypeStruct((_R * _CQ * _NW * _ZR,), jnp.float32),
        ),
        scratch_types=[
            [pltpu.VMEM((_K, _D), jnp.float32) for _ in range(2)],
            [pltpu.VMEM((_K,), jnp.int32) for _ in range(2)],
            [pltpu.VMEM((_K,), jnp.int32) for _ in range(2)],
            [pltpu.VMEM((_K,), jnp.int32) for _ in range(2)],
            pltpu.VMEM((_REM, _D), jnp.float32),
            pltpu.VMEM((_REM,), jnp.int32),
            pltpu.VMEM((_REM,), jnp.int32),
            pltpu.VMEM((_N,), jnp.float32),
            pltpu.VMEM_SHARED((_N, _D), jnp.float32),
            [pltpu.SemaphoreType.DMA for _ in range(2)],
            [pltpu.SemaphoreType.DMA for _ in range(2)],
            [pltpu.SemaphoreType.DMA for _ in range(2)],
        ],
        compiler_params=pltpu.CompilerParams(needs_layout_passes=False),
    )
    def body(x_hbm, srcs_hbm, dsts_hbm, zeros_hbm, agg_out, deg_out,
             rows, srcb, dstraw, dstb, rows_rem, src_rem, dst_rem, deg_v,
             agg_sh, sem_g, sem_s, sem_i):
        c = lax.axis_index("c")
        s = lax.axis_index("s")
        wid = s * _NC + c
        z16 = jnp.zeros((16,), jnp.float32)
        one16 = jnp.ones((16,), jnp.float32)

        for r in range(_R):
            ebase = pl.multiple_of(r * _E + wid * _EPT, 8)

            def _idx_start(k, b):
                off = pl.multiple_of(ebase + k * _K, 8)
                pltpu.async_copy(srcs_hbm.at[pl.ds(off, _K)], srcb[b],
                                 sem_i[b])
                pltpu.async_copy(dsts_hbm.at[pl.ds(off, _K)], dstraw[b],
                                 sem_i[b])

            def _idx_wait(k, b):
                off = pl.multiple_of(ebase + k * _K, 8)
                pltpu.make_async_copy(srcs_hbm.at[pl.ds(off, _K)], srcb[b],
                                      sem_i[b]).wait()
                pltpu.make_async_copy(dsts_hbm.at[pl.ds(off, _K)], dstraw[b],
                                      sem_i[b]).wait()

            def _gather(b):
                pltpu.async_copy(x_hbm.at[srcb[b]], rows[b], sem_g[b])

            def _gather_wait(b):
                pltpu.make_async_copy(x_hbm.at[srcb[b]], rows[b],
                                      sem_g[b]).wait()

            def _scatter_wait(b):
                pltpu.make_async_copy(rows[b], agg_sh.at[dstb[b]],
                                      sem_s[b]).wait()

            def _stage(b):
                for j in range(_K // 16):
                    dj = dstraw[b][pl.ds(j * 16, 16)]
                    plsc.addupdate_scatter(deg_v, [dj], one16)
                    dstb[b][pl.ds(j * 16, 16)] = dj

            for cq in range(2):
                q = cq * _NS + s

                @pl.when(q < _CQ)
                def _():
                    row = pl.multiple_of(q * _ZR, _ZR)
                    pltpu.sync_copy(zeros_hbm, agg_sh.at[pl.ds(row, _ZR)])

            def _zero_deg(i, carry):
                deg_v[pl.ds(i * 16, 16)] = z16
                return carry

            lax.fori_loop(0, _N // 16, _zero_deg, 0)
            plsc.subcore_barrier()

            _idx_start(0, 0)
            _idx_start(1, 1)
            _idx_wait(0, 0)
            _gather(0)

            def _group(g, carry):
                for b in range(2):
                    k = 2 * g + b
                    b2 = 1 - b
                    _gather_wait(b)
                    _stage(b)
                    pltpu.async_copy(rows[b], agg_sh.at[dstb[b]], sem_s[b],
                                     add=True)
                    if b == 0:

                        @pl.when(g + 1 < _NG)
                        def _():
                            _idx_start(k + 2, b)

                        @pl.when(g > 0)
                        def _():
                            _scatter_wait(b2)

                        _idx_wait(k + 1, b2)
                        _gather(b2)
                    else:

                        @pl.when(g + 1 < _NG)
                        def _():
                            _idx_start(k + 2, b)
                            _scatter_wait(b2)
                            _idx_wait(k + 1, b2)
                            _gather(b2)
                return carry

            lax.fori_loop(0, _NG, _group, 0)

            _scatter_wait(0)
            roff = pl.multiple_of(ebase + _FC * _K, 8)
            pltpu.sync_copy(srcs_hbm.at[pl.ds(roff, _REM)], src_rem)
            pltpu.sync_copy(dsts_hbm.at[pl.ds(roff, _REM)], dst_rem)
            pltpu.async_copy(x_hbm.at[src_rem], rows_rem, sem_g[0]).wait()
            dj = dst_rem[...]
            plsc.addupdate_scatter(deg_v, [dj], one16)
            pltpu.async_copy(rows_rem, agg_sh.at[dst_rem], sem_s[0], add=True)
            _scatter_wait(1)
            pltpu.make_async_copy(rows_rem, agg_sh.at[dst_rem],
                                  sem_s[0]).wait()
            plsc.subcore_barrier()

            for cq in range(2):
                q = cq * _NS + s

                @pl.when(q < _CQ)
                def _():
                    row = pl.multiple_of(q * _ZR, _ZR)
                    pltpu.sync_copy(agg_sh.at[pl.ds(row, _ZR)],
                                    agg_out.at[r, c, pl.ds(row, _ZR)])

            for q in range(_CQ):
                off = pl.multiple_of(((r * _CQ + q) * _NW + wid) * _ZR, 8)
                pltpu.sync_copy(deg_v.at[pl.ds(q * _ZR, _ZR)],
                                deg_out.at[pl.ds(off, _ZR)])

    return body(x, srcs, dsts, zeros)


_BN = 400
_NB = _N // _BN


def _tc_dense(x, agg, deg, w_self, w_neigh, bias):
    def body(x_ref, agg_ref, deg_ref, ws_ref, wn_ref, b_ref, out_ref):
        xb = x_ref[...]
        ws = ws_ref[...]
        wn = wn_ref[...]
        b = b_ref[...]
        acc = jnp.dot(xb, ws[0] + ws[1] + ws[2],
                      preferred_element_type=jnp.float32)
        degs = jnp.sum(deg_ref[...], axis=(1, 2))
        inv = 1.0 / jnp.maximum(degs, 1.0)
        ag = agg_ref[...]
        for r in range(_R):
            h = (ag[r, 0] + ag[r, 1]) * inv[r][:, None]
            acc = acc + jnp.dot(h, wn[r], preferred_element_type=jnp.float32)
        acc = acc + (b[0] + b[1] + b[2])[None, :]
        out_ref[...] = acc

    return pl.pallas_call(
        body,
        grid=(_NB,),
        in_specs=[
            pl.BlockSpec((_BN, _D), lambda i: (i, 0)),
            pl.BlockSpec((_R, _NC, _BN, _D), lambda i: (0, 0, i, 0)),
            pl.BlockSpec((_R, 1, _NW, _ZR), lambda i: (0, i, 0, 0)),
            pl.BlockSpec((_R, _D, _D), lambda i: (0, 0, 0)),
            pl.BlockSpec((_R, _D, _D), lambda i: (0, 0, 0)),
            pl.BlockSpec((_R, _D), lambda i: (0, 0)),
        ],
        out_specs=pl.BlockSpec((_BN, _D), lambda i: (i, 0)),
        out_shape=jax.ShapeDtypeStruct((_N, _D), jnp.float32),
    )(x, agg, deg, w_self, w_neigh, bias)


def kernel(x, edge_index_r0, edge_index_r1, edge_index_r2,
           W_self_r0, W_neigh_r0, b_r0,
           W_self_r1, W_neigh_r1, b_r1,
           W_self_r2, W_neigh_r2, b_r2):
    srcs = jnp.concatenate(
        [edge_index_r0[0], edge_index_r1[0], edge_index_r2[0]])
    dsts = jnp.concatenate(
        [edge_index_r0[1], edge_index_r1[1], edge_index_r2[1]])
    zeros = jnp.zeros((_ZR, _D), jnp.float32)
    agg, deg_flat = _sc_aggregate(x, srcs, dsts, zeros)
    deg = deg_flat.reshape(_R, _CQ, _NW, _ZR)
    w_self = jnp.stack([W_self_r0, W_self_r1, W_self_r2])
    w_neigh = jnp.stack([W_neigh_r0, W_neigh_r1, W_neigh_r2])
    bias = jnp.stack([b_r0, b_r1, b_r2])
    return _tc_dense(x, agg, deg, w_self, w_neigh, bias)
